# baseline (speedup 1.0000x reference)
import functools
import math

import jax
import jax.numpy as jnp
from jax import lax
from jax.experimental import pallas as pl
from jax.experimental.pallas import tpu as pltpu
from jax.experimental.pallas import tpu_sc as plsc

_V = 32000
_SMOOTHING = 0.1
_CONF = 1.0 - _SMOOTHING
_EPS = _SMOOTHING / _V

_ROWS_PER_BLOCK = 64

_LANES = 16
_CHUNK = 128


def _sum_block_kernel(x_ref, out_ref):
    out_ref[...] = jnp.sum(x_ref[...]).reshape(1, 1, 1)


def _make_sc_gather(n_rows):
    info = plsc.get_sparse_core_info()
    nc, ns = info.num_cores, info.num_subcores
    nw = nc * ns
    rows_per_w = n_rows // nw
    chunks = rows_per_w // _CHUNK
    vecs = _CHUNK // _LANES

    mesh = plsc.VectorSubcoreMesh(core_axis_name="c", subcore_axis_name="s")

    @functools.partial(
        pl.kernel,
        mesh=mesh,
        out_type=jax.ShapeDtypeStruct((nw * _LANES,), jnp.float32),
        scratch_types=[
            pltpu.VMEM((_CHUNK,), jnp.int32),
            pltpu.VMEM((_CHUNK,), jnp.int32),
            pltpu.VMEM((_CHUNK,), jnp.float32),
            pltpu.VMEM((_LANES,), jnp.float32),
            pltpu.SemaphoreType.DMA,
        ],
    )
    def sc_gather(x_flat_hbm, tgt_hbm, out_hbm, tgt_v, idx_v, vals_v, acc_v, sem):
        wid = lax.axis_index("s") * nc + lax.axis_index("c")
        base = wid * rows_per_w
        acc = jnp.zeros((_LANES,), jnp.float32)
        for c in range(chunks):
            cb = base + c * _CHUNK
            pltpu.sync_copy(tgt_hbm.at[pl.ds(cb, _CHUNK)], tgt_v)
            for j in range(vecs):
                t = tgt_v[pl.ds(j * _LANES, _LANES)]
                rows = (cb + j * _LANES) + lax.iota(jnp.int32, _LANES)
                idx_v[pl.ds(j * _LANES, _LANES)] = t + rows * _V
            pltpu.async_copy(x_flat_hbm.at[idx_v], vals_v, sem).wait()
            for j in range(vecs):
                acc = acc + vals_v[pl.ds(j * _LANES, _LANES)]
        acc_v[...] = acc
        pltpu.sync_copy(acc_v, out_hbm.at[pl.ds(wid * _LANES, _LANES)])

    return sc_gather


def kernel(x, target):
    n, v = x.shape
    r = _ROWS_PER_BLOCK
    g = n // r
    tc_partials = pl.pallas_call(
        _sum_block_kernel,
        grid=(g,),
        in_specs=[pl.BlockSpec((r, v), lambda i: (i, 0))],
        out_specs=pl.BlockSpec((1, 1, 1), lambda i: (i, 0, 0)),
        out_shape=jax.ShapeDtypeStruct((g, 1, 1), jnp.float32),
        compiler_params=pltpu.CompilerParams(
            dimension_semantics=("parallel",),
        ),
    )(x)
    sc_partials = _make_sc_gather(n)(x.reshape(-1), target.astype(jnp.int32))
    const = n * ((v - 1) * _EPS * math.log(_EPS) + _CONF * math.log(_CONF))
    return (
        jnp.float32(const)
        - jnp.float32(_EPS) * jnp.sum(tc_partials)
        - jnp.float32(_CONF - _EPS) * jnp.sum(sc_partials)
    )


# speedup vs baseline: 2.9901x; 2.9901x over previous
import functools
import math

import jax
import jax.numpy as jnp
from jax.experimental import pallas as pl
from jax.experimental.pallas import tpu as pltpu

_V = 32000
_SMOOTHING = 0.1
_CONF = 1.0 - _SMOOTHING
_EPS = _SMOOTHING / _V

_ROWS_PER_BLOCK = 128


def _loss_block_kernel(x_ref, tgt_ref, out_ref):
    x = x_ref[...]
    tgt = tgt_ref[0, 0, :]
    r, v = x.shape
    cols = jax.lax.broadcasted_iota(jnp.int32, (r, v), 1)
    hit = cols == tgt[:, None]
    gathered = jnp.sum(jnp.where(hit, x, 0.0))
    total = jnp.sum(x)
    partial = -_EPS * total - (_CONF - _EPS) * gathered
    out_ref[...] = partial.reshape(1, 1, 1)


@functools.partial(jax.jit, static_argnames=())
def kernel(x, target):
    n, v = x.shape
    r = _ROWS_PER_BLOCK
    g = n // r
    tgt3 = target.astype(jnp.int32).reshape(g, 1, r)
    partials = pl.pallas_call(
        _loss_block_kernel,
        grid=(g,),
        in_specs=[
            pl.BlockSpec((r, v), lambda i: (i, 0)),
            pl.BlockSpec((1, 1, r), lambda i: (i, 0, 0)),
        ],
        out_specs=pl.BlockSpec((1, 1, 1), lambda i: (i, 0, 0)),
        out_shape=jax.ShapeDtypeStruct((g, 1, 1), jnp.float32),
        compiler_params=pltpu.CompilerParams(
            dimension_semantics=("parallel",),
        ),
    )(x, tgt3)
    const = n * ((v - 1) * _EPS * math.log(_EPS) + _CONF * math.log(_CONF))
    return jnp.float32(const) + jnp.sum(partials)


# speedup vs baseline: 3.0489x; 1.0197x over previous
import functools
import math

import jax
import jax.numpy as jnp
from jax.experimental import pallas as pl
from jax.experimental.pallas import tpu as pltpu

_V = 32000
_SMOOTHING = 0.1
_CONF = 1.0 - _SMOOTHING
_EPS = _SMOOTHING / _V

_ROWS_PER_BLOCK = 256


def _loss_block_kernel(x_ref, tgt_ref, out_ref):
    x = x_ref[...]
    tgt = tgt_ref[0, 0, :]
    r, v = x.shape
    cols = jax.lax.broadcasted_iota(jnp.int32, (r, v), 1)
    hit = cols == tgt[:, None]
    gathered = jnp.sum(jnp.where(hit, x, 0.0))
    total = jnp.sum(x)
    partial = -_EPS * total - (_CONF - _EPS) * gathered
    out_ref[...] = partial.reshape(1, 1, 1)


@functools.partial(jax.jit, static_argnames=())
def kernel(x, target):
    n, v = x.shape
    r = _ROWS_PER_BLOCK
    g = n // r
    tgt3 = target.astype(jnp.int32).reshape(g, 1, r)
    partials = pl.pallas_call(
        _loss_block_kernel,
        grid=(g,),
        in_specs=[
            pl.BlockSpec((r, v), lambda i: (i, 0)),
            pl.BlockSpec((1, 1, r), lambda i: (i, 0, 0)),
        ],
        out_specs=pl.BlockSpec((1, 1, 1), lambda i: (i, 0, 0)),
        out_shape=jax.ShapeDtypeStruct((g, 1, 1), jnp.float32),
        compiler_params=pltpu.CompilerParams(
            dimension_semantics=("parallel",),
            vmem_limit_bytes=128 * 1024 * 1024,
        ),
    )(x, tgt3)
    const = n * ((v - 1) * _EPS * math.log(_EPS) + _CONF * math.log(_CONF))
    return jnp.float32(const) + jnp.sum(partials)
